# baseline (speedup 1.0000x reference)
import functools

import jax
import jax.numpy as jnp
from jax import lax
from jax.experimental import pallas as pl
from jax.experimental.pallas import tpu as pltpu
from jax.experimental.pallas import tpu_sc as plsc

B = 16384
D = 32
V = 1000000
NC = 2
NS = 16
NW = NC * NS
B_PER_W = B // NW
NBUF = 16
L = 16

_mesh = plsc.VectorSubcoreMesh(core_axis_name="c", subcore_axis_name="s")


@functools.partial(
    pl.kernel,
    mesh=_mesh,
    out_type=jax.ShapeDtypeStruct((D, B), jnp.float32),
    scratch_types=[
        pltpu.VMEM((B_PER_W + L,), jnp.int32),
        pltpu.VMEM((NBUF, D, 128), jnp.float32),
        pltpu.VMEM((D, B_PER_W), jnp.float32),
        [pltpu.SemaphoreType.DMA] * NBUF,
    ],
    compiler_params=pltpu.CompilerParams(needs_layout_passes=False),
)
def _gather_kernel(table_hbm, idx_hbm, out_hbm, idx_v, tiles_v, out_v, sems):
    wid = lax.axis_index("s") * NC + lax.axis_index("c")
    base = wid * B_PER_W
    pltpu.sync_copy(idx_hbm.at[pl.ds(base, B_PER_W)],
                    idx_v.at[pl.ds(0, B_PER_W)])

    def fire(j, b):
        i = idx_v[pl.ds(j, L)][0]
        off = pl.multiple_of(i & ~127, 128)
        pltpu.async_copy(
            table_hbm.at[:, pl.ds(off, 128)],
            tiles_v.at[b],
            sems[b],
        )

    for b in range(NBUF):
        fire(b, b)

    iota = lax.iota(jnp.int32, L)
    rows0 = iota
    rows1 = iota + L

    def body(g, carry):
        for b in range(NBUF):
            j = g * NBUF + b
            pltpu.make_async_copy(
                table_hbm.at[:, pl.ds(0, 128)], tiles_v.at[b], sems[b]
            ).wait()
            w = idx_v[pl.ds(j, L)][0] & 127
            col = jnp.full((L,), w, jnp.int32)
            jcol = jnp.full((L,), j, jnp.int32)
            vals0 = plsc.load_gather(tiles_v.at[b], [rows0, col])
            vals1 = plsc.load_gather(tiles_v.at[b], [rows1, col])
            plsc.store_scatter(out_v, [rows0, jcol], vals0)
            plsc.store_scatter(out_v, [rows1, jcol], vals1)
            @pl.when(g < B_PER_W // NBUF - 1)
            def _():
                fire(j + NBUF, b)
        return carry

    lax.fori_loop(0, B_PER_W // NBUF, body, 0)
    pltpu.sync_copy(out_v, out_hbm.at[:, pl.ds(base, B_PER_W)])


def kernel(node_embeddings, object_indices):
    table_t = node_embeddings.T
    idx = object_indices.astype(jnp.int32)
    out_t = _gather_kernel(table_t, idx)
    return out_t.T


# speedup vs baseline: 1.0400x; 1.0400x over previous
import functools

import jax
import jax.numpy as jnp
from jax import lax
from jax.experimental import pallas as pl
from jax.experimental.pallas import tpu as pltpu
from jax.experimental.pallas import tpu_sc as plsc

B = 16384
D = 32
V = 1000000
NC = 2
NS = 16
NW = NC * NS
L = 16

CW = 512
NCHUNK_MAIN = 61
CPW = 244
DUMP = B

_mesh = plsc.VectorSubcoreMesh(core_axis_name="c", subcore_axis_name="s")


@functools.partial(
    pl.kernel,
    mesh=_mesh,
    out_type=jax.ShapeDtypeStruct((B + 1, 128), jnp.float32),
    scratch_types=[
        pltpu.VMEM((B,), jnp.int32),
        pltpu.VMEM((B + L,), jnp.int32),
        pltpu.VMEM((B + L,), jnp.int32),
        pltpu.VMEM((B + L,), jnp.int32),
        pltpu.VMEM((B + L,), jnp.int32),
        pltpu.VMEM((2, D, CW), jnp.float32),
        pltpu.VMEM((2, L, 128), jnp.float32),
        pltpu.VMEM((2, L), jnp.int32),
        [pltpu.SemaphoreType.DMA] * 2,
        pltpu.SemaphoreType.DMA,
    ],
    compiler_params=pltpu.CompilerParams(needs_layout_passes=False),
)
def _gather_kernel(table_hbm, idx_hbm, out_hbm, idx_all, my_i, my_j, cl_i,
                   cl_j, sbuf, rowb, posb, fsems, ssem):
    wid = lax.axis_index("s") * NC + lax.axis_index("c")
    last = (wid == NW - 1).astype(jnp.int32)
    nchunk = NCHUNK_MAIN + 2 * last
    iota = lax.iota(jnp.int32, L)
    rows0 = iota
    rows1 = iota + L

    pltpu.sync_copy(idx_hbm, idx_all)

    def drain_one_scatter():
        pltpu.make_async_copy(
            table_hbm.at[pl.ds(0, L), pl.ds(0, 128)], rowb.at[0], ssem
        ).wait()

    lo = wid * CPW
    hi = lo + CPW + 5 * last

    def prefilter(k, off):
        iv = idx_all[pl.ds(k * L, L)]
        jv = iota + k * L
        c = lax.shift_right_logical(iv, 7)
        m = (c >= lo) & (c < hi)
        cnt = plsc.all_reduce_population_count(m)[0]
        plsc.store_compressed(my_i.at[pl.ds(off, L)], iv, mask=m)
        plsc.store_compressed(my_j.at[pl.ds(off, L)], jv, mask=m)
        return off + cnt

    count = lax.fori_loop(0, B // L, prefilter, 0)
    nstrip = lax.shift_right_logical(count + L - 1, 4)

    def chunk_off(g):
        main = (wid * NCHUNK_MAIN + g) * CW
        return jnp.where(g == NCHUNK_MAIN, 999424,
                         jnp.where(g == NCHUNK_MAIN + 1, 999552, main))

    def fire_chunk(g, b):
        off = pl.multiple_of(chunk_off(g), 128)
        pltpu.async_copy(
            table_hbm.at[:, pl.ds(off, CW)], sbuf.at[b], fsems[b]
        )

    fire_chunk(0, 0)

    def process(g, b, carry):
        off = chunk_off(g)
        base_c = lax.shift_right_logical(off, 7)
        sel_lo = jnp.where(g == NCHUNK_MAIN + 1, base_c + 3, base_c)
        sel_hi = jnp.where(g < nchunk, base_c + 4, base_c)

        def scan(k, coff):
            iv = my_i[pl.ds(k * L, L)]
            jv = my_j[pl.ds(k * L, L)]
            c = lax.shift_right_logical(iv, 7)
            m = (c >= sel_lo) & (c < sel_hi) & (iota + k * L < count)
            cnt = plsc.all_reduce_population_count(m)[0]
            plsc.store_compressed(cl_i.at[pl.ds(coff, L)], iv, mask=m)
            plsc.store_compressed(cl_j.at[pl.ds(coff, L)], jv, mask=m)
            return coff + cnt

        ccount = lax.fori_loop(0, nstrip, scan, 0)

        def extract(e, ec):
            ectr, posvec = ec
            i = cl_i[pl.ds(e, L)][0]
            j = cl_j[pl.ds(e, L)][0]
            colw = jnp.full((L,), i - off, jnp.int32)
            vals0 = plsc.load_gather(sbuf.at[b], [rows0, colw])
            vals1 = plsc.load_gather(sbuf.at[b], [rows1, colw])
            m = ectr & (L - 1)
            bb = lax.shift_right_logical(ectr, 4) & 1
            row = rowb.at[bb].at[m]
            row[pl.ds(0, L)] = vals0
            row[pl.ds(L, L)] = vals1
            posvec = jnp.where(iota == m, j, posvec)

            @pl.when(m == L - 1)
            def _():
                posb.at[bb][...] = posvec

                @pl.when(ectr >= 2 * L)
                def _():
                    drain_one_scatter()

                pltpu.async_copy(rowb.at[bb], out_hbm.at[posb.at[bb]], ssem)

            posvec = jnp.where(m == L - 1, jnp.full((L,), DUMP, jnp.int32),
                               posvec)
            return ectr + 1, posvec

        return lax.fori_loop(0, ccount, extract, carry)

    def pair_body(p, carry):
        for bs in range(2):
            g = 2 * p + bs

            @pl.when(g + 1 < nchunk)
            def _():
                fire_chunk(g + 1, 1 - bs)

            @pl.when(g < nchunk)
            def _():
                pltpu.make_async_copy(
                    table_hbm.at[:, pl.ds(0, CW)], sbuf.at[bs], fsems[bs]
                ).wait()

            carry = process(g, bs, carry)
        return carry

    posvec0 = jnp.full((L,), DUMP, jnp.int32)
    ectr, posvec = lax.fori_loop(0, (NCHUNK_MAIN + 3) // 2, pair_body,
                                 (0, posvec0))

    nfull = lax.shift_right_logical(ectr, 4)

    @pl.when((ectr & (L - 1)) != 0)
    def _():
        bb = nfull & 1
        posb.at[bb][...] = posvec

        @pl.when(nfull >= 2)
        def _():
            drain_one_scatter()

        pltpu.async_copy(rowb.at[bb], out_hbm.at[posb.at[bb]], ssem)

    nfired = lax.shift_right_logical(ectr + L - 1, 4)
    ndrain = jnp.minimum(nfired, 2)

    def drain(_, carry):
        drain_one_scatter()
        return carry

    lax.fori_loop(0, ndrain, drain, 0)


def kernel(node_embeddings, object_indices):
    table_t = node_embeddings.T
    idx = object_indices.astype(jnp.int32)
    out4 = _gather_kernel(table_t, idx)
    return out4[:B, :D]
